# baseline (speedup 1.0000x reference)
import functools

import jax
import jax.numpy as jnp
from jax import lax
from jax.experimental import pallas as pl
from jax.experimental.pallas import tpu as pltpu
from jax.experimental.pallas import tpu_sc as plsc

NC = 2
NS = 16
L = 16
NW = NC * NS
K = 128
DEGW = 16


def _mesh():
    return plsc.VectorSubcoreMesh(
        core_axis_name="c", subcore_axis_name="s",
        num_cores=NC, num_subcores=NS)


@functools.lru_cache(maxsize=None)
def _make_kdeg(NPAD, C):
    @functools.partial(
        pl.kernel,
        out_type=jax.ShapeDtypeStruct((NC, NPAD, DEGW), jnp.float32),
        mesh=_mesh(),
        scratch_types=[
            pltpu.VMEM((C, K), jnp.int32),
            pltpu.VMEM((K, DEGW), jnp.float32),
            pltpu.VMEM_SHARED((NPAD, DEGW), jnp.float32),
        ],
    )
    def kdeg(dst_hbm, zeros_hbm, out_hbm, dst_v, ones_v, acc):
        c = lax.axis_index("c")
        s = lax.axis_index("s")
        w = s * NC + c

        def fill(r, carry):
            ones_v[r, pl.ds(0, L)] = jnp.ones((L,), jnp.float32)
            return carry
        lax.fori_loop(0, K, fill, 0)

        @pl.when(s == 0)
        def _():
            pltpu.sync_copy(zeros_hbm, acc)
        pltpu.sync_copy(dst_hbm.at[w], dst_v)
        plsc.subcore_barrier()

        def body(j, carry):
            pltpu.sync_copy(ones_v, acc.at[dst_v.at[j]], add=True)
            return carry
        lax.fori_loop(0, C, body, 0)

        plsc.subcore_barrier()

        @pl.when(s == 0)
        def _():
            pltpu.sync_copy(acc, out_hbm.at[c])

    return kdeg


@functools.lru_cache(maxsize=None)
def _make_kagg(NPAD, D, C, KA, NB):
    NI = 2 * NB
    assert C % NI == 0 and C >= NI

    @functools.partial(
        pl.kernel,
        out_type=jax.ShapeDtypeStruct((NC, NPAD, D), jnp.float32),
        mesh=_mesh(),
        scratch_types=[
            pltpu.VMEM((2 * NI, KA), jnp.int32),
            [pltpu.VMEM((KA, D), jnp.float32) for _ in range(NB)],
            [pltpu.SemaphoreType.DMA for _ in range(NB)],
            [pltpu.SemaphoreType.DMA for _ in range(NB)],
            [pltpu.SemaphoreType.DMA for _ in range(NI)],
            pltpu.VMEM_SHARED((NPAD, D), jnp.float32),
        ],
    )
    def kagg(y_hbm, sd_hbm, zeros_hbm, out_hbm,
             idx_v, rows_v, gsem, ssem, isem, acc):
        c = lax.axis_index("c")
        s = lax.axis_index("s")
        w = s * NC + c

        @pl.when(s == 0)
        def _():
            pltpu.sync_copy(zeros_hbm, acc)

        for i in range(NI):
            pltpu.async_copy(sd_hbm.at[w, i],
                             idx_v.at[pl.ds(2 * i, 2)], isem[i])
        plsc.subcore_barrier()

        for b in range(NB):
            pltpu.make_async_copy(sd_hbm.at[w, b],
                                  idx_v.at[pl.ds(2 * b, 2)], isem[b]).wait()
            pltpu.async_copy(y_hbm.at[idx_v.at[2 * b]], rows_v[b], gsem[b])

        def group(g, carry):
            for q in range(NI):
                b = q % NB
                jj = g * NI + q
                pltpu.make_async_copy(y_hbm.at[idx_v.at[2 * q]],
                                      rows_v[b], gsem[b]).wait()
                pltpu.async_copy(rows_v[b], acc.at[idx_v.at[2 * q + 1]],
                                 ssem[b], add=True).wait()

                @pl.when(jj + NI < C)
                def _():
                    pltpu.async_copy(sd_hbm.at[w, jj + NI],
                                     idx_v.at[pl.ds(2 * q, 2)], isem[q])

                @pl.when(jj + NB < C)
                def _():
                    qn = (q + NB) % NI
                    pltpu.make_async_copy(sd_hbm.at[w, jj + NB],
                                          idx_v.at[pl.ds(2 * qn, 2)],
                                          isem[qn]).wait()
                    pltpu.async_copy(y_hbm.at[idx_v.at[2 * qn]],
                                     rows_v[b], gsem[b])
            return carry
        lax.fori_loop(0, C // NI, group, 0)

        plsc.subcore_barrier()

        @pl.when(s == 0)
        def _():
            pltpu.sync_copy(acc, out_hbm.at[c])

    return kagg


def _mm(x, W, B=2048):
    NPAD, D = x.shape

    def body(x_ref, w_ref, o_ref):
        o_ref[...] = jnp.dot(x_ref[...], w_ref[...],
                             preferred_element_type=jnp.float32)

    return pl.pallas_call(
        body,
        grid=(NPAD // B,),
        in_specs=[pl.BlockSpec((B, D), lambda i: (i, 0)),
                  pl.BlockSpec((D, D), lambda i: (0, 0))],
        out_specs=pl.BlockSpec((B, D), lambda i: (i, 0)),
        out_shape=jax.ShapeDtypeStruct((NPAD, D), jnp.float32),
    )(x, W)


def _prep(d0, d1, xw, B=2048):
    NPAD, D = xw.shape

    def body(d0_ref, d1_ref, xw_ref, y_ref, dis_ref):
        deg = d0_ref[:, 0:1] + d1_ref[:, 0:1] + 1.0
        dis = lax.rsqrt(deg)
        y_ref[...] = dis * xw_ref[...]
        dis_ref[...] = jnp.broadcast_to(dis, xw_ref.shape)

    return pl.pallas_call(
        body,
        grid=(NPAD // B,),
        in_specs=[pl.BlockSpec((B, DEGW), lambda i: (i, 0)),
                  pl.BlockSpec((B, DEGW), lambda i: (i, 0)),
                  pl.BlockSpec((B, D), lambda i: (i, 0))],
        out_specs=[pl.BlockSpec((B, D), lambda i: (i, 0)),
                   pl.BlockSpec((B, D), lambda i: (i, 0))],
        out_shape=[jax.ShapeDtypeStruct((NPAD, D), jnp.float32),
                   jax.ShapeDtypeStruct((NPAD, D), jnp.float32)],
    )(d0, d1, xw)


def _mid(a0, a1, y1, dis, W2, b1, B=2048):
    NPAD, D = y1.shape

    def body(a0_ref, a1_ref, y1_ref, dis_ref, w_ref, b_ref, y2_ref):
        h = dis_ref[...] * (a0_ref[...] + a1_ref[...] + y1_ref[...]) + b_ref[...]
        h = jnp.maximum(h, 0.0)
        y2_ref[...] = dis_ref[...] * jnp.dot(h, w_ref[...],
                                             preferred_element_type=jnp.float32)

    return pl.pallas_call(
        body,
        grid=(NPAD // B,),
        in_specs=[pl.BlockSpec((B, D), lambda i: (i, 0)),
                  pl.BlockSpec((B, D), lambda i: (i, 0)),
                  pl.BlockSpec((B, D), lambda i: (i, 0)),
                  pl.BlockSpec((B, D), lambda i: (i, 0)),
                  pl.BlockSpec((D, D), lambda i: (0, 0)),
                  pl.BlockSpec((1, D), lambda i: (0, 0))],
        out_specs=pl.BlockSpec((B, D), lambda i: (i, 0)),
        out_shape=jax.ShapeDtypeStruct((NPAD, D), jnp.float32),
    )(a0, a1, y1, dis, W2, b1)


def _fin(a0, a1, y2, dis, b2, N, B=2000):
    NPAD, D = y2.shape

    def body(a0_ref, a1_ref, y2_ref, dis_ref, b_ref, o_ref):
        o_ref[...] = (dis_ref[...] * (a0_ref[...] + a1_ref[...] + y2_ref[...])
                      + b_ref[...])

    return pl.pallas_call(
        body,
        grid=(N // B,),
        in_specs=[pl.BlockSpec((B, D), lambda i: (i, 0)),
                  pl.BlockSpec((B, D), lambda i: (i, 0)),
                  pl.BlockSpec((B, D), lambda i: (i, 0)),
                  pl.BlockSpec((B, D), lambda i: (i, 0)),
                  pl.BlockSpec((1, D), lambda i: (0, 0))],
        out_specs=pl.BlockSpec((B, D), lambda i: (i, 0)),
        out_shape=jax.ShapeDtypeStruct((N, D), jnp.float32),
    )(a0, a1, y2, dis, b2)


def kernel(features, edges, edges2, edge_features, additional_feature,
           W1, b1, W2, b2):
    N, D = features.shape
    E = edges2.shape[1]

    NPAD = -(-(N + 1) // (NS * K)) * (NS * K)
    KA, NB = 64, 3
    Cd = -(-E // (NW * K))
    Ca = -(-(-(-E // (NW * KA))) // (2 * NB)) * (2 * NB)
    EPD = Cd * K * NW
    EPA = Ca * KA * NW

    src = edges2[0]
    dst = edges2[1]
    padd = jnp.full((EPD - E,), N, jnp.int32)
    pada = jnp.full((EPA - E,), N, jnp.int32)
    dst3d = jnp.concatenate([dst, padd]).reshape(NW, Cd, K)
    src3a = jnp.concatenate([src, pada]).reshape(NW, Ca, KA)
    dst3a = jnp.concatenate([dst, pada]).reshape(NW, Ca, KA)
    sd3 = jnp.stack([src3a, dst3a], axis=2)
    x_pad = jnp.pad(features, ((0, NPAD - N), (0, 0)))
    b1r = b1.reshape(1, D)
    b2r = b2.reshape(1, D)

    kdeg = _make_kdeg(NPAD, Cd)
    kagg = _make_kagg(NPAD, D, Ca, KA, NB)
    zdeg = jnp.zeros((NPAD, DEGW), jnp.float32)
    zagg = jnp.zeros((NPAD, D), jnp.float32)

    deg_p = kdeg(dst3d, zdeg)
    xw1 = _mm(x_pad, W1)
    y1, dis = _prep(deg_p[0], deg_p[1], xw1)
    ag1 = kagg(y1, sd3, zagg)
    y2 = _mid(ag1[0], ag1[1], y1, dis, W2, b1r)
    ag2 = kagg(y2, sd3, zagg)
    return _fin(ag2[0], ag2[1], y2, dis, b2r, N)


# speedup vs baseline: 1.3709x; 1.3709x over previous
import functools

import jax
import jax.numpy as jnp
from jax import lax
from jax.experimental import pallas as pl
from jax.experimental.pallas import tpu as pltpu
from jax.experimental.pallas import tpu_sc as plsc

NC = 2
NS = 16
L = 16
NW = NC * NS
K = 128
DEGW = 16


def _mesh():
    return plsc.VectorSubcoreMesh(
        core_axis_name="c", subcore_axis_name="s",
        num_cores=NC, num_subcores=NS)


@functools.lru_cache(maxsize=None)
def _make_kdeg(NPAD, C):
    @functools.partial(
        pl.kernel,
        out_type=jax.ShapeDtypeStruct((NC, NPAD, DEGW), jnp.float32),
        mesh=_mesh(),
        scratch_types=[
            pltpu.VMEM((C, K), jnp.int32),
            pltpu.VMEM((K, DEGW), jnp.float32),
            pltpu.VMEM_SHARED((NPAD, DEGW), jnp.float32),
        ],
    )
    def kdeg(dst_hbm, zeros_hbm, out_hbm, dst_v, ones_v, acc):
        c = lax.axis_index("c")
        s = lax.axis_index("s")
        w = s * NC + c

        def fill(r, carry):
            ones_v[r, pl.ds(0, L)] = jnp.ones((L,), jnp.float32)
            return carry
        lax.fori_loop(0, K, fill, 0)

        @pl.when(s == 0)
        def _():
            pltpu.sync_copy(zeros_hbm, acc)
        pltpu.sync_copy(dst_hbm.at[w], dst_v)
        plsc.subcore_barrier()

        def body(j, carry):
            pltpu.sync_copy(ones_v, acc.at[dst_v.at[j]], add=True)
            return carry
        lax.fori_loop(0, C, body, 0)

        plsc.subcore_barrier()

        @pl.when(s == 0)
        def _():
            pltpu.sync_copy(acc, out_hbm.at[c])

    return kdeg


@functools.lru_cache(maxsize=None)
def _make_kagg(NPAD, D, C, KA, RB):
    GD = RB // 2
    NI = 2 * RB
    assert C % NI == 0 and C >= NI

    @functools.partial(
        pl.kernel,
        out_type=jax.ShapeDtypeStruct((NC, NPAD, D), jnp.float32),
        mesh=_mesh(),
        scratch_types=[
            pltpu.VMEM((2 * NI, KA), jnp.int32),
            [pltpu.VMEM((KA, D), jnp.float32) for _ in range(RB)],
            [pltpu.SemaphoreType.DMA for _ in range(RB)],
            [pltpu.SemaphoreType.DMA for _ in range(RB)],
            [pltpu.SemaphoreType.DMA for _ in range(NI)],
            pltpu.VMEM_SHARED((NPAD, D), jnp.float32),
        ],
    )
    def kagg(y_hbm, sd_hbm, zeros_hbm, out_hbm,
             idx_v, rows_v, gsem, ssem, isem, acc):
        c = lax.axis_index("c")
        s = lax.axis_index("s")
        w = s * NC + c

        @pl.when(s == 0)
        def _():
            pltpu.sync_copy(zeros_hbm, acc)

        for i in range(NI):
            pltpu.async_copy(sd_hbm.at[w, i],
                             idx_v.at[pl.ds(2 * i, 2)], isem[i])
        plsc.subcore_barrier()

        def wait_idx(m):
            p = m % NI
            pltpu.make_async_copy(sd_hbm.at[w, 0],
                                  idx_v.at[pl.ds(2 * p, 2)], isem[p]).wait()

        def fire_gather(m):
            pltpu.async_copy(y_hbm.at[idx_v.at[2 * (m % NI)]],
                             rows_v[m % RB], gsem[m % RB])

        def wait_gather(m):
            pltpu.make_async_copy(y_hbm.at[idx_v.at[2 * (m % NI)]],
                                  rows_v[m % RB], gsem[m % RB]).wait()

        def fire_scatter(m):
            pltpu.async_copy(rows_v[m % RB],
                             acc.at[idx_v.at[2 * (m % NI) + 1]],
                             ssem[m % RB], add=True)

        def wait_scatter(m):
            pltpu.make_async_copy(rows_v[m % RB],
                                  acc.at[idx_v.at[2 * (m % NI) + 1]],
                                  ssem[m % RB]).wait()

        for m in range(GD):
            wait_idx(m)
            fire_gather(m)

        def group(g, carry):
            for q in range(NI):
                jj = g * NI + q
                wait_gather(q)
                fire_scatter(q)
                wait_scatter(q)

                @pl.when(jj + NI < C)
                def _():
                    pltpu.async_copy(sd_hbm.at[w, jj + NI],
                                     idx_v.at[pl.ds(2 * q, 2)],
                                     isem[q])

                @pl.when(jj + GD < C)
                def _():
                    wait_idx(q + GD)
                    fire_gather(q + GD)
            return carry
        lax.fori_loop(0, C // NI, group, 0)

        plsc.subcore_barrier()

        @pl.when(s == 0)
        def _():
            pltpu.sync_copy(acc, out_hbm.at[c])

    return kagg


def _mm(x, W, B=2048):
    NPAD, D = x.shape

    def body(x_ref, w_ref, o_ref):
        o_ref[...] = jnp.dot(x_ref[...], w_ref[...],
                             preferred_element_type=jnp.float32)

    return pl.pallas_call(
        body,
        grid=(NPAD // B,),
        in_specs=[pl.BlockSpec((B, D), lambda i: (i, 0)),
                  pl.BlockSpec((D, D), lambda i: (0, 0))],
        out_specs=pl.BlockSpec((B, D), lambda i: (i, 0)),
        out_shape=jax.ShapeDtypeStruct((NPAD, D), jnp.float32),
    )(x, W)


def _prep(d0, d1, xw, B=2048):
    NPAD, D = xw.shape

    def body(d0_ref, d1_ref, xw_ref, y_ref, dis_ref):
        deg = d0_ref[:, 0:1] + d1_ref[:, 0:1] + 1.0
        dis = lax.rsqrt(deg)
        y_ref[...] = dis * xw_ref[...]
        dis_ref[...] = jnp.broadcast_to(dis, xw_ref.shape)

    return pl.pallas_call(
        body,
        grid=(NPAD // B,),
        in_specs=[pl.BlockSpec((B, DEGW), lambda i: (i, 0)),
                  pl.BlockSpec((B, DEGW), lambda i: (i, 0)),
                  pl.BlockSpec((B, D), lambda i: (i, 0))],
        out_specs=[pl.BlockSpec((B, D), lambda i: (i, 0)),
                   pl.BlockSpec((B, D), lambda i: (i, 0))],
        out_shape=[jax.ShapeDtypeStruct((NPAD, D), jnp.float32),
                   jax.ShapeDtypeStruct((NPAD, D), jnp.float32)],
    )(d0, d1, xw)


def _mid(a0, a1, y1, dis, W2, b1, B=2048):
    NPAD, D = y1.shape

    def body(a0_ref, a1_ref, y1_ref, dis_ref, w_ref, b_ref, y2_ref):
        h = dis_ref[...] * (a0_ref[...] + a1_ref[...] + y1_ref[...]) + b_ref[...]
        h = jnp.maximum(h, 0.0)
        y2_ref[...] = dis_ref[...] * jnp.dot(h, w_ref[...],
                                             preferred_element_type=jnp.float32)

    return pl.pallas_call(
        body,
        grid=(NPAD // B,),
        in_specs=[pl.BlockSpec((B, D), lambda i: (i, 0)),
                  pl.BlockSpec((B, D), lambda i: (i, 0)),
                  pl.BlockSpec((B, D), lambda i: (i, 0)),
                  pl.BlockSpec((B, D), lambda i: (i, 0)),
                  pl.BlockSpec((D, D), lambda i: (0, 0)),
                  pl.BlockSpec((1, D), lambda i: (0, 0))],
        out_specs=pl.BlockSpec((B, D), lambda i: (i, 0)),
        out_shape=jax.ShapeDtypeStruct((NPAD, D), jnp.float32),
    )(a0, a1, y1, dis, W2, b1)


def _fin(a0, a1, y2, dis, b2, N, B=2000):
    NPAD, D = y2.shape

    def body(a0_ref, a1_ref, y2_ref, dis_ref, b_ref, o_ref):
        o_ref[...] = (dis_ref[...] * (a0_ref[...] + a1_ref[...] + y2_ref[...])
                      + b_ref[...])

    return pl.pallas_call(
        body,
        grid=(N // B,),
        in_specs=[pl.BlockSpec((B, D), lambda i: (i, 0)),
                  pl.BlockSpec((B, D), lambda i: (i, 0)),
                  pl.BlockSpec((B, D), lambda i: (i, 0)),
                  pl.BlockSpec((B, D), lambda i: (i, 0)),
                  pl.BlockSpec((1, D), lambda i: (0, 0))],
        out_specs=pl.BlockSpec((B, D), lambda i: (i, 0)),
        out_shape=jax.ShapeDtypeStruct((N, D), jnp.float32),
    )(a0, a1, y2, dis, b2)


def kernel(features, edges, edges2, edge_features, additional_feature,
           W1, b1, W2, b2):
    N, D = features.shape
    E = edges2.shape[1]

    NPAD = -(-(N + 1) // (NS * K)) * (NS * K)
    KA, RB = 64, 4
    Cd = -(-E // (NW * K))
    Ca = -(-(-(-E // (NW * KA))) // (2 * RB)) * (2 * RB)
    EPD = Cd * K * NW
    EPA = Ca * KA * NW

    src = edges2[0]
    dst = edges2[1]
    padd = jnp.full((EPD - E,), N, jnp.int32)
    pada = jnp.full((EPA - E,), N, jnp.int32)
    dst3d = jnp.concatenate([dst, padd]).reshape(NW, Cd, K)
    src3a = jnp.concatenate([src, pada]).reshape(NW, Ca, KA)
    dst3a = jnp.concatenate([dst, pada]).reshape(NW, Ca, KA)
    sd3 = jnp.stack([src3a, dst3a], axis=2)
    x_pad = jnp.pad(features, ((0, NPAD - N), (0, 0)))
    b1r = b1.reshape(1, D)
    b2r = b2.reshape(1, D)

    kdeg = _make_kdeg(NPAD, Cd)
    kagg = _make_kagg(NPAD, D, Ca, KA, RB)
    zdeg = jnp.zeros((NPAD, DEGW), jnp.float32)
    zagg = jnp.zeros((NPAD, D), jnp.float32)

    deg_p = kdeg(dst3d, zdeg)
    xw1 = _mm(x_pad, W1)
    y1, dis = _prep(deg_p[0], deg_p[1], xw1)
    ag1 = kagg(y1, sd3, zagg)
    y2 = _mid(ag1[0], ag1[1], y1, dis, W2, b1r)
    ag2 = kagg(y2, sd3, zagg)
    return _fin(ag2[0], ag2[1], y2, dis, b2r, N)


# speedup vs baseline: 2.1266x; 1.5513x over previous
import functools

import jax
import jax.numpy as jnp
from jax import lax
from jax.experimental import pallas as pl
from jax.experimental.pallas import tpu as pltpu
from jax.experimental.pallas import tpu_sc as plsc

NC = 2
NS = 16
L = 16
NW = NC * NS
K = 128
DEGW = 16


def _mesh():
    return plsc.VectorSubcoreMesh(
        core_axis_name="c", subcore_axis_name="s",
        num_cores=NC, num_subcores=NS)


@functools.lru_cache(maxsize=None)
def _make_kdeg(NPAD, C):
    @functools.partial(
        pl.kernel,
        out_type=jax.ShapeDtypeStruct((NW, NPAD), jnp.float32),
        mesh=_mesh(),
        scratch_types=[
            pltpu.VMEM((C, K), jnp.int32),
            pltpu.VMEM((NPAD + L,), jnp.float32),
        ],
    )
    def kdeg(dst_hbm, out_hbm, dst_v, cnt_v):
        c = lax.axis_index("c")
        s = lax.axis_index("s")
        w = s * NC + c

        pltpu.sync_copy(dst_hbm.at[w], dst_v)

        def zfill(i, carry):
            cnt_v[pl.ds(i * L, L)] = jnp.zeros((L,), jnp.float32)
            return carry
        lax.fori_loop(0, (NPAD + L) // L, zfill, 0)

        onehot0 = (lax.iota(jnp.int32, L) == 0).astype(jnp.float32)

        def row(j, carry):
            def grp(k, carry2):
                vecidx = dst_v[j, pl.ds(k * L, L)]
                for i in range(L):
                    idx = vecidx[i]
                    cnt_v[pl.ds(idx, L)] = cnt_v[pl.ds(idx, L)] + onehot0
                return carry2
            return lax.fori_loop(0, K // L, grp, carry)
        lax.fori_loop(0, C, row, 0)

        pltpu.sync_copy(cnt_v.at[pl.ds(0, NPAD)], out_hbm.at[w])

    return kdeg


@functools.lru_cache(maxsize=None)
def _make_kagg(NPAD, D, C):
    @functools.partial(
        pl.kernel,
        out_type=jax.ShapeDtypeStruct((NC, NPAD, D), jnp.float32),
        mesh=_mesh(),
        scratch_types=[
            pltpu.VMEM((C, K), jnp.int32),
            pltpu.VMEM((C, K), jnp.int32),
            pltpu.VMEM((K, D), jnp.float32),
            pltpu.VMEM_SHARED((NPAD, D), jnp.float32),
            pltpu.SemaphoreType.DMA,
        ],
    )
    def kagg(y_hbm, src_hbm, dst_hbm, zeros_hbm, out_hbm,
             src_v, dst_v, rows_v, acc, sem):
        c = lax.axis_index("c")
        s = lax.axis_index("s")
        w = s * NC + c

        @pl.when(s == 0)
        def _():
            pltpu.sync_copy(zeros_hbm, acc)
        pltpu.sync_copy(src_hbm.at[w], src_v)
        pltpu.sync_copy(dst_hbm.at[w], dst_v)
        plsc.subcore_barrier()

        def body(j, carry):
            pltpu.async_copy(y_hbm.at[src_v.at[j]], rows_v, sem).wait()
            pltpu.sync_copy(rows_v, acc.at[dst_v.at[j]], add=True)
            return carry
        lax.fori_loop(0, C, body, 0)

        plsc.subcore_barrier()

        @pl.when(s == 0)
        def _():
            pltpu.sync_copy(acc, out_hbm.at[c])

    return kagg


def _mm(x, W, B=2048):
    NPAD, D = x.shape

    def body(x_ref, w_ref, o_ref):
        o_ref[...] = jnp.dot(x_ref[...], w_ref[...],
                             preferred_element_type=jnp.float32)

    return pl.pallas_call(
        body,
        grid=(NPAD // B,),
        in_specs=[pl.BlockSpec((B, D), lambda i: (i, 0)),
                  pl.BlockSpec((D, D), lambda i: (0, 0))],
        out_specs=pl.BlockSpec((B, D), lambda i: (i, 0)),
        out_shape=jax.ShapeDtypeStruct((NPAD, D), jnp.float32),
    )(x, W)


def _prep(d0, d1, xw, B=2048):
    NPAD, D = xw.shape

    def body(d0_ref, d1_ref, xw_ref, y_ref, dis_ref):
        deg = d0_ref[:, 0:1] + d1_ref[:, 0:1] + 1.0
        dis = lax.rsqrt(deg)
        y_ref[...] = dis * xw_ref[...]
        dis_ref[...] = jnp.broadcast_to(dis, xw_ref.shape)

    return pl.pallas_call(
        body,
        grid=(NPAD // B,),
        in_specs=[pl.BlockSpec((B, D), lambda i: (i, 0)),
                  pl.BlockSpec((B, D), lambda i: (i, 0)),
                  pl.BlockSpec((B, D), lambda i: (i, 0))],
        out_specs=[pl.BlockSpec((B, D), lambda i: (i, 0)),
                   pl.BlockSpec((B, D), lambda i: (i, 0))],
        out_shape=[jax.ShapeDtypeStruct((NPAD, D), jnp.float32),
                   jax.ShapeDtypeStruct((NPAD, D), jnp.float32)],
    )(d0, d1, xw)


def _mid(a0, a1, y1, dis, W2, b1, B=2048):
    NPAD, D = y1.shape

    def body(a0_ref, a1_ref, y1_ref, dis_ref, w_ref, b_ref, y2_ref):
        h = dis_ref[...] * (a0_ref[...] + a1_ref[...] + y1_ref[...]) + b_ref[...]
        h = jnp.maximum(h, 0.0)
        y2_ref[...] = dis_ref[...] * jnp.dot(h, w_ref[...],
                                             preferred_element_type=jnp.float32)

    return pl.pallas_call(
        body,
        grid=(NPAD // B,),
        in_specs=[pl.BlockSpec((B, D), lambda i: (i, 0)),
                  pl.BlockSpec((B, D), lambda i: (i, 0)),
                  pl.BlockSpec((B, D), lambda i: (i, 0)),
                  pl.BlockSpec((B, D), lambda i: (i, 0)),
                  pl.BlockSpec((D, D), lambda i: (0, 0)),
                  pl.BlockSpec((1, D), lambda i: (0, 0))],
        out_specs=pl.BlockSpec((B, D), lambda i: (i, 0)),
        out_shape=jax.ShapeDtypeStruct((NPAD, D), jnp.float32),
    )(a0, a1, y1, dis, W2, b1)


def _fin(a0, a1, y2, dis, b2, N, B=2000):
    NPAD, D = y2.shape

    def body(a0_ref, a1_ref, y2_ref, dis_ref, b_ref, o_ref):
        o_ref[...] = (dis_ref[...] * (a0_ref[...] + a1_ref[...] + y2_ref[...])
                      + b_ref[...])

    return pl.pallas_call(
        body,
        grid=(N // B,),
        in_specs=[pl.BlockSpec((B, D), lambda i: (i, 0)),
                  pl.BlockSpec((B, D), lambda i: (i, 0)),
                  pl.BlockSpec((B, D), lambda i: (i, 0)),
                  pl.BlockSpec((B, D), lambda i: (i, 0)),
                  pl.BlockSpec((1, D), lambda i: (0, 0))],
        out_specs=pl.BlockSpec((B, D), lambda i: (i, 0)),
        out_shape=jax.ShapeDtypeStruct((N, D), jnp.float32),
    )(a0, a1, y2, dis, b2)


def kernel(features, edges, edges2, edge_features, additional_feature,
           W1, b1, W2, b2):
    N, D = features.shape
    E = edges2.shape[1]

    NPAD = -(-(N + 1) // (NS * K)) * (NS * K)
    C = -(-E // (NW * K))
    EPAD = C * K * NW

    src = edges2[0]
    dst = edges2[1]
    padv = (N + (jnp.arange(EPAD - E, dtype=jnp.int32)
                 % (NPAD - N))).astype(jnp.int32)
    src3 = jnp.concatenate([src, padv]).reshape(NW, C, K)
    dst3 = jnp.concatenate([dst, padv]).reshape(NW, C, K)
    x_pad = jnp.pad(features, ((0, NPAD - N), (0, 0)))
    b1r = b1.reshape(1, D)
    b2r = b2.reshape(1, D)

    kagg = _make_kagg(NPAD, D, C)
    zagg = jnp.zeros((NPAD, D), jnp.float32)
    ones_t = jnp.ones((NPAD, D), jnp.float32)

    deg_p = kagg(ones_t, src3, dst3, zagg)
    xw1 = _mm(x_pad, W1)
    y1, dis = _prep(deg_p[0], deg_p[1], xw1)
    ag1 = kagg(y1, src3, dst3, zagg)
    y2 = _mid(ag1[0], ag1[1], y1, dis, W2, b1r)
    ag2 = kagg(y2, src3, dst3, zagg)
    return _fin(ag2[0], ag2[1], y2, dis, b2r, N)
